# baseline (speedup 1.0000x reference)
import numpy as np
import jax
import jax.numpy as jnp
from jax.experimental import pallas as pl
from jax.experimental.pallas import tpu as pltpu

V = 100000
ROWS = 512
P = 8
STEPS = ROWS // P
W = 512
NFULL = V // W
TAIL = V - NFULL * W

_KU = (np.uint32(1832780943), np.uint32(270669613))
_KS = (np.uint32(64467757), np.uint32(2916123636))

_C999 = np.float32(0.999 - 1e-12)
_EPS = np.float32(1e-12)
_TINY = np.float32(np.finfo(np.float32).tiny)
_G_MAX = np.float32(6.91)
_NEG_INF = np.float32(-np.inf)
_BIG = np.int32(2**31 - 1)

_UNROLL = 13

_ROT_A = (13, 15, 26, 6)
_ROT_B = (17, 29, 16, 24)


def _threefry_bits(k1, k2, lo):
    k3 = np.uint32(np.uint32(k1) ^ np.uint32(k2) ^ np.uint32(0x1BD11BDA))
    ks = [k1, k2, k3]
    x0 = jnp.full_like(lo, k1)
    x1 = lo + k2
    rots = [_ROT_A, _ROT_B]
    for i in range(5):
        for r in rots[0]:
            x0 = x0 + x1
            x1 = (x1 << np.uint32(r)) | (x1 >> np.uint32(32 - r))
            x1 = x1 ^ x0
        x0 = x0 + ks[1]
        x1 = x1 + ks[2] + np.uint32(i + 1)
        ks = ks[1:] + ks[:1]
        rots = rots[1:] + rots[:1]
    return x0 ^ x1


def _bits_to_u01(bits):
    fb = (bits >> np.uint32(9)) | np.uint32(0x3F800000)
    return jax.lax.bitcast_convert_type(fb, jnp.float32) - np.float32(1.0)


def _stgs_body(x_ref, y1_ref, y2_ref, ids_ref):
    i = pl.program_id(0)
    rowbase = ((jnp.uint32(i * P * V))
               + jax.lax.broadcasted_iota(jnp.uint32, (P, 1), 0) * jnp.uint32(V))
    m = jnp.max(x_ref[...], axis=1, keepdims=True) + _G_MAX

    def chunk(off, xw, w):
        col = jax.lax.broadcasted_iota(jnp.uint32, (P, w), 1)
        idx = rowbase + (col + off)
        u = _bits_to_u01(_threefry_bits(_KU[0], _KU[1], idx)) * _C999 + _EPS
        a = -jnp.log(u)
        e = jnp.exp(xw - m) / a
        u2 = jnp.maximum(_TINY, _bits_to_u01(_threefry_bits(_KS[0], _KS[1], idx)))
        val = e / (-jnp.log(u2))
        return e, val

    col_i32 = jax.lax.broadcasted_iota(jnp.int32, (P, W), 1)

    def body_a(j, carry):
        s_run = carry
        off = j * W
        col = jax.lax.broadcasted_iota(jnp.uint32, (P, W), 1)
        idx = rowbase + (col + jnp.uint32(off))
        u = _bits_to_u01(_threefry_bits(_KU[0], _KU[1], idx)) * _C999 + _EPS
        e = jnp.exp(x_ref[:, pl.ds(off, W)] - m) / (-jnp.log(u))
        y1_ref[:, pl.ds(off, W)] = e
        return s_run + e

    def body_b(j, carry):
        runmax, runidx = carry
        off = j * W
        col = jax.lax.broadcasted_iota(jnp.uint32, (P, W), 1)
        idx = rowbase + (col + jnp.uint32(off))
        u2 = jnp.maximum(_TINY, _bits_to_u01(_threefry_bits(_KS[0], _KS[1], idx)))
        val = y1_ref[:, pl.ds(off, W)] / (-jnp.log(u2))
        upd = val > runmax
        return (jnp.where(upd, val, runmax),
                jnp.where(upd, col_i32 + off, runidx))

    s0 = jnp.zeros((P, W), jnp.float32)
    mx0 = jnp.full((P, W), _NEG_INF, jnp.float32)
    id0 = jnp.zeros((P, W), jnp.int32)
    s_run = jax.lax.fori_loop(0, NFULL, body_a, s0, unroll=_UNROLL)
    runmax, runidx = jax.lax.fori_loop(0, NFULL, body_b, (mx0, id0),
                                       unroll=_UNROLL)

    toff = NFULL * W
    xw_t = x_ref[:, pl.ds(toff, TAIL)]
    e_t, val_t = chunk(jnp.uint32(toff), xw_t, TAIL)
    y1_ref[:, pl.ds(toff, TAIL)] = e_t

    s = (jnp.sum(s_run, axis=1, keepdims=True)
         + jnp.sum(e_t, axis=1, keepdims=True))

    rowmax = jnp.max(runmax, axis=1, keepdims=True)
    rid = jnp.min(jnp.where(runmax == rowmax, runidx, _BIG),
                  axis=1, keepdims=True)
    tmax = jnp.max(val_t, axis=1, keepdims=True)
    tcol = jax.lax.broadcasted_iota(jnp.int32, (P, TAIL), 1) + np.int32(toff)
    tid = jnp.min(jnp.where(val_t == tmax, tcol, _BIG), axis=1, keepdims=True)
    ids = jnp.where(tmax > rowmax, tid, rid).astype(jnp.float32)
    ids_ref[...] = ids.reshape(1, P, 1)

    rs = np.float32(1.0) / s

    def norm_body(j, _):
        off = j * W
        v = y1_ref[:, pl.ds(off, W)] * rs
        y1_ref[:, pl.ds(off, W)] = v
        y2_ref[:, pl.ds(off, W)] = v
        return 0

    jax.lax.fori_loop(0, NFULL, norm_body, 0)
    v_t = e_t * rs
    y1_ref[:, pl.ds(toff, TAIL)] = v_t
    y2_ref[:, pl.ds(toff, TAIL)] = v_t


def kernel(x):
    xf = x.reshape(ROWS, V)
    y1, y2, ids = pl.pallas_call(
        _stgs_body,
        grid=(STEPS,),
        in_specs=[pl.BlockSpec((P, V), lambda i: (i, 0))],
        out_specs=[
            pl.BlockSpec((P, V), lambda i: (i, 0)),
            pl.BlockSpec((P, V), lambda i: (i, 0)),
            pl.BlockSpec((1, P, 1), lambda i: (i, 0, 0)),
        ],
        out_shape=[
            jax.ShapeDtypeStruct((ROWS, V), jnp.float32),
            jax.ShapeDtypeStruct((ROWS, V), jnp.float32),
            jax.ShapeDtypeStruct((STEPS, P, 1), jnp.float32),
        ],
        compiler_params=pltpu.CompilerParams(
            dimension_semantics=("arbitrary",),
        ),
    )(xf)
    diff_ids = ids.reshape(64, 8)
    y_soft = y1.reshape(64, 8, V)
    one_hot = y2.reshape(64, 8, V)
    eff_temperature = jnp.ones((1,), jnp.float32)
    return (diff_ids, one_hot, eff_temperature, y_soft)


# speedup vs baseline: 1.0358x; 1.0358x over previous
import numpy as np
import jax
import jax.numpy as jnp
from jax.experimental import pallas as pl
from jax.experimental.pallas import tpu as pltpu

V = 100000
ROWS = 512
P = 8
STEPS = ROWS // P
W = 512
NFULL = V // W
TAIL = V - NFULL * W

_KU = (np.uint32(1832780943), np.uint32(270669613))
_KS = (np.uint32(64467757), np.uint32(2916123636))

_C999 = np.float32(0.999 - 1e-12)
_EPS = np.float32(1e-12)
_TINY = np.float32(np.finfo(np.float32).tiny)
_G_MAX = np.float32(6.91)
_NEG_INF = np.float32(-np.inf)
_BIG = np.int32(2**31 - 1)

_UNROLL = 15

_ROT_A = (13, 15, 26, 6)
_ROT_B = (17, 29, 16, 24)


def _threefry_bits(k1, k2, lo):
    k3 = np.uint32(np.uint32(k1) ^ np.uint32(k2) ^ np.uint32(0x1BD11BDA))
    ks = [k1, k2, k3]
    x0 = jnp.full_like(lo, k1)
    x1 = lo + k2
    rots = [_ROT_A, _ROT_B]
    for i in range(5):
        for r in rots[0]:
            x0 = x0 + x1
            x1 = (x1 << np.uint32(r)) | (x1 >> np.uint32(32 - r))
            x1 = x1 ^ x0
        x0 = x0 + ks[1]
        x1 = x1 + ks[2] + np.uint32(i + 1)
        ks = ks[1:] + ks[:1]
        rots = rots[1:] + rots[:1]
    return x0 ^ x1


def _bits_to_u01(bits):
    fb = (bits >> np.uint32(9)) | np.uint32(0x3F800000)
    return jax.lax.bitcast_convert_type(fb, jnp.float32) - np.float32(1.0)


def _stgs_body(x_ref, y1_ref, y2_ref, ids_ref):
    i = pl.program_id(0)
    rowbase = ((jnp.uint32(i * P * V))
               + jax.lax.broadcasted_iota(jnp.uint32, (P, 1), 0) * jnp.uint32(V))
    m = jnp.max(x_ref[...], axis=1, keepdims=True) + _G_MAX

    def chunk(off, xw, w):
        col = jax.lax.broadcasted_iota(jnp.uint32, (P, w), 1)
        idx = rowbase + (col + off)
        u = _bits_to_u01(_threefry_bits(_KU[0], _KU[1], idx)) * _C999 + _EPS
        a = -jnp.log(u)
        e = jnp.exp(xw - m) / a
        u2 = jnp.maximum(_TINY, _bits_to_u01(_threefry_bits(_KS[0], _KS[1], idx)))
        val = e / (-jnp.log(u2))
        return e, val

    col_i32 = jax.lax.broadcasted_iota(jnp.int32, (P, W), 1)

    def body(j, carry):
        s_run, runmax, runidx = carry
        off = j * W
        xw = x_ref[:, pl.ds(off, W)]
        e, val = chunk(jnp.uint32(off), xw, W)
        y1_ref[:, pl.ds(off, W)] = e
        upd = val > runmax
        return (s_run + e,
                jnp.where(upd, val, runmax),
                jnp.where(upd, col_i32 + off, runidx))

    s0 = jnp.zeros((P, W), jnp.float32)
    mx0 = jnp.full((P, W), _NEG_INF, jnp.float32)
    id0 = jnp.zeros((P, W), jnp.int32)
    s_run, runmax, runidx = jax.lax.fori_loop(0, NFULL, body, (s0, mx0, id0),
                                              unroll=_UNROLL)

    toff = NFULL * W
    xw_t = x_ref[:, pl.ds(toff, TAIL)]
    e_t, val_t = chunk(jnp.uint32(toff), xw_t, TAIL)
    y1_ref[:, pl.ds(toff, TAIL)] = e_t

    s = (jnp.sum(s_run, axis=1, keepdims=True)
         + jnp.sum(e_t, axis=1, keepdims=True))

    rowmax = jnp.max(runmax, axis=1, keepdims=True)
    rid = jnp.min(jnp.where(runmax == rowmax, runidx, _BIG),
                  axis=1, keepdims=True)
    tmax = jnp.max(val_t, axis=1, keepdims=True)
    tcol = jax.lax.broadcasted_iota(jnp.int32, (P, TAIL), 1) + np.int32(toff)
    tid = jnp.min(jnp.where(val_t == tmax, tcol, _BIG), axis=1, keepdims=True)
    ids = jnp.where(tmax > rowmax, tid, rid).astype(jnp.float32)
    ids_ref[...] = ids.reshape(1, P, 1)

    rs = np.float32(1.0) / s

    def norm_body(j, _):
        off = j * W
        v = y1_ref[:, pl.ds(off, W)] * rs
        y1_ref[:, pl.ds(off, W)] = v
        y2_ref[:, pl.ds(off, W)] = v
        return 0

    jax.lax.fori_loop(0, NFULL, norm_body, 0, unroll=8)
    v_t = e_t * rs
    y1_ref[:, pl.ds(toff, TAIL)] = v_t
    y2_ref[:, pl.ds(toff, TAIL)] = v_t


def kernel(x):
    xf = x.reshape(ROWS, V)
    y1, y2, ids = pl.pallas_call(
        _stgs_body,
        grid=(STEPS,),
        in_specs=[pl.BlockSpec((P, V), lambda i: (i, 0))],
        out_specs=[
            pl.BlockSpec((P, V), lambda i: (i, 0)),
            pl.BlockSpec((P, V), lambda i: (i, 0)),
            pl.BlockSpec((1, P, 1), lambda i: (i, 0, 0)),
        ],
        out_shape=[
            jax.ShapeDtypeStruct((ROWS, V), jnp.float32),
            jax.ShapeDtypeStruct((ROWS, V), jnp.float32),
            jax.ShapeDtypeStruct((STEPS, P, 1), jnp.float32),
        ],
        compiler_params=pltpu.CompilerParams(
            dimension_semantics=("arbitrary",),
        ),
    )(xf)
    diff_ids = ids.reshape(64, 8)
    y_soft = y1.reshape(64, 8, V)
    one_hot = y2.reshape(64, 8, V)
    eff_temperature = jnp.ones((1,), jnp.float32)
    return (diff_ids, one_hot, eff_temperature, y_soft)


# speedup vs baseline: 1.0387x; 1.0028x over previous
import numpy as np
import jax
import jax.numpy as jnp
from jax.experimental import pallas as pl
from jax.experimental.pallas import tpu as pltpu

V = 100000
ROWS = 512
P = 8
STEPS = ROWS // P
W = 512
NFULL = V // W
TAIL = V - NFULL * W

_KU = (np.uint32(1832780943), np.uint32(270669613))
_KS = (np.uint32(64467757), np.uint32(2916123636))

_C999 = np.float32(0.999 - 1e-12)
_EPS = np.float32(1e-12)
_TINY = np.float32(np.finfo(np.float32).tiny)
_G_MAX = np.float32(6.91)
_NEG_INF = np.float32(-np.inf)
_BIG = np.int32(2**31 - 1)

_UNROLL = 20

_ROT_A = (13, 15, 26, 6)
_ROT_B = (17, 29, 16, 24)


def _threefry_bits(k1, k2, lo):
    k3 = np.uint32(np.uint32(k1) ^ np.uint32(k2) ^ np.uint32(0x1BD11BDA))
    ks = [k1, k2, k3]
    x0 = jnp.full_like(lo, k1)
    x1 = lo + k2
    rots = [_ROT_A, _ROT_B]
    for i in range(5):
        for r in rots[0]:
            x0 = x0 + x1
            x1 = (x1 << np.uint32(r)) | (x1 >> np.uint32(32 - r))
            x1 = x1 ^ x0
        x0 = x0 + ks[1]
        x1 = x1 + ks[2] + np.uint32(i + 1)
        ks = ks[1:] + ks[:1]
        rots = rots[1:] + rots[:1]
    return x0 ^ x1


def _bits_to_u01(bits):
    fb = (bits >> np.uint32(9)) | np.uint32(0x3F800000)
    return jax.lax.bitcast_convert_type(fb, jnp.float32) - np.float32(1.0)


def _stgs_body(x_ref, y1_ref, y2_ref, ids_ref):
    i = pl.program_id(0)
    rowbase = ((jnp.uint32(i * P * V))
               + jax.lax.broadcasted_iota(jnp.uint32, (P, 1), 0) * jnp.uint32(V))
    m = jnp.max(x_ref[...], axis=1, keepdims=True) + _G_MAX

    def chunk(off, xw, w):
        col = jax.lax.broadcasted_iota(jnp.uint32, (P, w), 1)
        idx = rowbase + (col + off)
        u = _bits_to_u01(_threefry_bits(_KU[0], _KU[1], idx)) * _C999 + _EPS
        a = -jnp.log(u)
        e = jnp.exp(xw - m) / a
        u2 = jnp.maximum(_TINY, _bits_to_u01(_threefry_bits(_KS[0], _KS[1], idx)))
        val = e / (-jnp.log(u2))
        return e, val

    col_i32 = jax.lax.broadcasted_iota(jnp.int32, (P, W), 1)

    def body(j, carry):
        s_run, runmax, runidx = carry
        off = j * W
        xw = x_ref[:, pl.ds(off, W)]
        e, val = chunk(jnp.uint32(off), xw, W)
        y1_ref[:, pl.ds(off, W)] = e
        upd = val > runmax
        return (s_run + e,
                jnp.where(upd, val, runmax),
                jnp.where(upd, col_i32 + off, runidx))

    s0 = jnp.zeros((P, W), jnp.float32)
    mx0 = jnp.full((P, W), _NEG_INF, jnp.float32)
    id0 = jnp.zeros((P, W), jnp.int32)
    s_run, runmax, runidx = jax.lax.fori_loop(0, NFULL, body, (s0, mx0, id0),
                                              unroll=_UNROLL)

    toff = NFULL * W
    xw_t = x_ref[:, pl.ds(toff, TAIL)]
    e_t, val_t = chunk(jnp.uint32(toff), xw_t, TAIL)
    y1_ref[:, pl.ds(toff, TAIL)] = e_t

    s = (jnp.sum(s_run, axis=1, keepdims=True)
         + jnp.sum(e_t, axis=1, keepdims=True))

    rowmax = jnp.max(runmax, axis=1, keepdims=True)
    rid = jnp.min(jnp.where(runmax == rowmax, runidx, _BIG),
                  axis=1, keepdims=True)
    tmax = jnp.max(val_t, axis=1, keepdims=True)
    tcol = jax.lax.broadcasted_iota(jnp.int32, (P, TAIL), 1) + np.int32(toff)
    tid = jnp.min(jnp.where(val_t == tmax, tcol, _BIG), axis=1, keepdims=True)
    ids = jnp.where(tmax > rowmax, tid, rid).astype(jnp.float32)
    ids_ref[...] = ids.reshape(1, P, 1)

    rs = np.float32(1.0) / s

    def norm_body(j, _):
        off = j * W
        v = y1_ref[:, pl.ds(off, W)] * rs
        y1_ref[:, pl.ds(off, W)] = v
        y2_ref[:, pl.ds(off, W)] = v
        return 0

    jax.lax.fori_loop(0, NFULL, norm_body, 0, unroll=8)
    v_t = e_t * rs
    y1_ref[:, pl.ds(toff, TAIL)] = v_t
    y2_ref[:, pl.ds(toff, TAIL)] = v_t


def kernel(x):
    xf = x.reshape(ROWS, V)
    y1, y2, ids = pl.pallas_call(
        _stgs_body,
        grid=(STEPS,),
        in_specs=[pl.BlockSpec((P, V), lambda i: (i, 0))],
        out_specs=[
            pl.BlockSpec((P, V), lambda i: (i, 0)),
            pl.BlockSpec((P, V), lambda i: (i, 0)),
            pl.BlockSpec((1, P, 1), lambda i: (i, 0, 0)),
        ],
        out_shape=[
            jax.ShapeDtypeStruct((ROWS, V), jnp.float32),
            jax.ShapeDtypeStruct((ROWS, V), jnp.float32),
            jax.ShapeDtypeStruct((STEPS, P, 1), jnp.float32),
        ],
        compiler_params=pltpu.CompilerParams(
            dimension_semantics=("arbitrary",),
        ),
    )(xf)
    diff_ids = ids.reshape(64, 8)
    y_soft = y1.reshape(64, 8, V)
    one_hot = y2.reshape(64, 8, V)
    eff_temperature = jnp.ones((1,), jnp.float32)
    return (diff_ids, one_hot, eff_temperature, y_soft)


# speedup vs baseline: 1.0407x; 1.0019x over previous
import numpy as np
import jax
import jax.numpy as jnp
from jax.experimental import pallas as pl
from jax.experimental.pallas import tpu as pltpu

V = 100000
ROWS = 512
P = 8
STEPS = ROWS // P
W = 512
NFULL = V // W
TAIL = V - NFULL * W

_KU = (np.uint32(1832780943), np.uint32(270669613))
_KS = (np.uint32(64467757), np.uint32(2916123636))

_C999 = np.float32(0.999 - 1e-12)
_EPS = np.float32(1e-12)
_TINY = np.float32(np.finfo(np.float32).tiny)
_G_MAX = np.float32(6.91)
_NEG_INF = np.float32(-np.inf)
_BIG = np.int32(2**31 - 1)

_UNROLL = 39

_ROT_A = (13, 15, 26, 6)
_ROT_B = (17, 29, 16, 24)


def _threefry_bits(k1, k2, lo):
    k3 = np.uint32(np.uint32(k1) ^ np.uint32(k2) ^ np.uint32(0x1BD11BDA))
    ks = [k1, k2, k3]
    x0 = jnp.full_like(lo, k1)
    x1 = lo + k2
    rots = [_ROT_A, _ROT_B]
    for i in range(5):
        for r in rots[0]:
            x0 = x0 + x1
            x1 = (x1 << np.uint32(r)) | (x1 >> np.uint32(32 - r))
            x1 = x1 ^ x0
        x0 = x0 + ks[1]
        x1 = x1 + ks[2] + np.uint32(i + 1)
        ks = ks[1:] + ks[:1]
        rots = rots[1:] + rots[:1]
    return x0 ^ x1


def _bits_to_u01(bits):
    fb = (bits >> np.uint32(9)) | np.uint32(0x3F800000)
    return jax.lax.bitcast_convert_type(fb, jnp.float32) - np.float32(1.0)


def _stgs_body(x_ref, y1_ref, y2_ref, ids_ref):
    i = pl.program_id(0)
    rowbase = ((jnp.uint32(i * P * V))
               + jax.lax.broadcasted_iota(jnp.uint32, (P, 1), 0) * jnp.uint32(V))
    m = jnp.max(x_ref[...], axis=1, keepdims=True) + _G_MAX

    def chunk(off, xw, w):
        col = jax.lax.broadcasted_iota(jnp.uint32, (P, w), 1)
        idx = rowbase + (col + off)
        u = _bits_to_u01(_threefry_bits(_KU[0], _KU[1], idx)) * _C999 + _EPS
        a = -jnp.log(u)
        e = jnp.exp(xw - m) / a
        u2 = jnp.maximum(_TINY, _bits_to_u01(_threefry_bits(_KS[0], _KS[1], idx)))
        val = e / (-jnp.log(u2))
        return e, val

    col_i32 = jax.lax.broadcasted_iota(jnp.int32, (P, W), 1)

    def body(j, carry):
        s_run, runmax, runidx = carry
        off = j * W
        xw = x_ref[:, pl.ds(off, W)]
        e, val = chunk(jnp.uint32(off), xw, W)
        y1_ref[:, pl.ds(off, W)] = e
        upd = val > runmax
        return (s_run + e,
                jnp.where(upd, val, runmax),
                jnp.where(upd, col_i32 + off, runidx))

    s0 = jnp.zeros((P, W), jnp.float32)
    mx0 = jnp.full((P, W), _NEG_INF, jnp.float32)
    id0 = jnp.zeros((P, W), jnp.int32)
    s_run, runmax, runidx = jax.lax.fori_loop(0, NFULL, body, (s0, mx0, id0),
                                              unroll=_UNROLL)

    toff = NFULL * W
    xw_t = x_ref[:, pl.ds(toff, TAIL)]
    e_t, val_t = chunk(jnp.uint32(toff), xw_t, TAIL)
    y1_ref[:, pl.ds(toff, TAIL)] = e_t

    s = (jnp.sum(s_run, axis=1, keepdims=True)
         + jnp.sum(e_t, axis=1, keepdims=True))

    rowmax = jnp.max(runmax, axis=1, keepdims=True)
    rid = jnp.min(jnp.where(runmax == rowmax, runidx, _BIG),
                  axis=1, keepdims=True)
    tmax = jnp.max(val_t, axis=1, keepdims=True)
    tcol = jax.lax.broadcasted_iota(jnp.int32, (P, TAIL), 1) + np.int32(toff)
    tid = jnp.min(jnp.where(val_t == tmax, tcol, _BIG), axis=1, keepdims=True)
    ids = jnp.where(tmax > rowmax, tid, rid).astype(jnp.float32)
    ids_ref[...] = ids.reshape(1, P, 1)

    rs = np.float32(1.0) / s

    def norm_body(j, _):
        off = j * W
        v = y1_ref[:, pl.ds(off, W)] * rs
        y1_ref[:, pl.ds(off, W)] = v
        y2_ref[:, pl.ds(off, W)] = v
        return 0

    jax.lax.fori_loop(0, NFULL, norm_body, 0, unroll=8)
    v_t = e_t * rs
    y1_ref[:, pl.ds(toff, TAIL)] = v_t
    y2_ref[:, pl.ds(toff, TAIL)] = v_t


def kernel(x):
    xf = x.reshape(ROWS, V)
    y1, y2, ids = pl.pallas_call(
        _stgs_body,
        grid=(STEPS,),
        in_specs=[pl.BlockSpec((P, V), lambda i: (i, 0))],
        out_specs=[
            pl.BlockSpec((P, V), lambda i: (i, 0)),
            pl.BlockSpec((P, V), lambda i: (i, 0)),
            pl.BlockSpec((1, P, 1), lambda i: (i, 0, 0)),
        ],
        out_shape=[
            jax.ShapeDtypeStruct((ROWS, V), jnp.float32),
            jax.ShapeDtypeStruct((ROWS, V), jnp.float32),
            jax.ShapeDtypeStruct((STEPS, P, 1), jnp.float32),
        ],
        compiler_params=pltpu.CompilerParams(
            dimension_semantics=("arbitrary",),
        ),
    )(xf)
    diff_ids = ids.reshape(64, 8)
    y_soft = y1.reshape(64, 8, V)
    one_hot = y2.reshape(64, 8, V)
    eff_temperature = jnp.ones((1,), jnp.float32)
    return (diff_ids, one_hot, eff_temperature, y_soft)
